# Initial kernel scaffold; baseline (speedup 1.0000x reference)
import jax
import jax.numpy as jnp
from jax.experimental import pallas as pl


def kernel(x, edge_index, W1, b1, g1, be1, W2, b2, g2, be2, W3, b3):
    raise NotImplementedError("write your pallas kernel here")


# speedup vs baseline: 10.9985x; 10.9985x over previous
import functools

import jax
import jax.numpy as jnp
from jax import lax
from jax.experimental import pallas as pl
from jax.experimental.pallas import tpu as pltpu
from jax.experimental.pallas import tpu_sc as plsc

N = 10000
D = 128
NC = 2
NS = 16
NW = NC * NS
CH = 128
NCH = 79
SLOTS = NW * NCH * CH
SROWS = 640
NPAD = NS * SROWS

_mesh = plsc.VectorSubcoreMesh(core_axis_name="c", subcore_axis_name="s")


def _deg_body(dstI_hbm, z1_hbm, out_hbm, dst_v, ones_v, dbuf, deg_sh):
    cid = lax.axis_index("c")
    sid = lax.axis_index("s")
    t = cid * NS + sid
    soff = pl.multiple_of(sid * SROWS, 8)
    for j in range(CH // 16):
        ones_v[pl.ds(j * 16, 16)] = jnp.ones((16,), jnp.float32)
    pltpu.sync_copy(z1_hbm, dbuf)
    pltpu.sync_copy(dbuf, deg_sh.at[pl.ds(soff, SROWS)])
    pltpu.sync_copy(dstI_hbm.at[t], dst_v)
    plsc.subcore_barrier()

    def body(c, carry):
        pltpu.sync_copy(ones_v, deg_sh.at[dst_v.at[c]], add=True)
        return carry

    lax.fori_loop(0, NCH, body, 0)
    plsc.subcore_barrier()
    ooff = pl.multiple_of(cid * NPAD + sid * SROWS, 8)
    pltpu.sync_copy(deg_sh.at[pl.ds(soff, SROWS)], dbuf)
    pltpu.sync_copy(dbuf, out_hbm.at[pl.ds(ooff, SROWS)])


_deg_call = pl.kernel(
    _deg_body,
    out_type=jax.ShapeDtypeStruct((NC * NPAD,), jnp.float32),
    mesh=_mesh,
    scratch_types=[
        pltpu.VMEM((NCH, CH), jnp.int32),
        pltpu.VMEM((CH,), jnp.float32),
        pltpu.VMEM((SROWS,), jnp.float32),
        pltpu.VMEM_SHARED((NPAD,), jnp.float32),
    ],
)


def _agg_body(h_hbm, srcI_hbm, dstI_hbm, z2_hbm, out_hbm,
              src_v, dst_v, buf, agg_sh):
    cid = lax.axis_index("c")
    sid = lax.axis_index("s")
    t = cid * NS + sid
    pltpu.sync_copy(z2_hbm, buf)
    for k in range(SROWS // CH):
        koff = pl.multiple_of(sid * SROWS + k * CH, 8)
        pltpu.sync_copy(buf, agg_sh.at[pl.ds(koff, CH)])
    pltpu.sync_copy(srcI_hbm.at[t], src_v)
    pltpu.sync_copy(dstI_hbm.at[t], dst_v)
    plsc.subcore_barrier()

    def body(c, carry):
        pltpu.sync_copy(h_hbm.at[src_v.at[c]], buf)
        pltpu.sync_copy(buf, agg_sh.at[dst_v.at[c]], add=True)
        return carry

    lax.fori_loop(0, NCH, body, 0)
    plsc.subcore_barrier()
    for k in range(SROWS // CH):
        koff = pl.multiple_of(sid * SROWS + k * CH, 8)
        pltpu.sync_copy(agg_sh.at[pl.ds(koff, CH)], buf)
        pltpu.sync_copy(buf, out_hbm.at[cid, pl.ds(koff, CH)])


_agg_call = pl.kernel(
    _agg_body,
    out_type=jax.ShapeDtypeStruct((NC, NPAD, D), jnp.float32),
    mesh=_mesh,
    scratch_types=[
        pltpu.VMEM((NCH, CH), jnp.int32),
        pltpu.VMEM((NCH, CH), jnp.int32),
        pltpu.VMEM((CH, D), jnp.float32),
        pltpu.VMEM_SHARED((NPAD, D), jnp.float32),
    ],
)


def _tca_body(dd_ref, x_ref, w_ref, h1s_ref, dinv_ref):
    deg = dd_ref[:, 0:1] + dd_ref[:, 1:2] + 1.0
    dinv = lax.rsqrt(deg)
    h = jnp.dot(x_ref[...], w_ref[...], preferred_element_type=jnp.float32)
    h1s_ref[...] = h * dinv
    dinv_ref[...] = dinv


def _tca(dd, x, w):
    return pl.pallas_call(
        _tca_body,
        out_shape=(jax.ShapeDtypeStruct((N, D), jnp.float32),
                   jax.ShapeDtypeStruct((N, 1), jnp.float32)),
    )(dd, x, w)


def _tcb_body(aggp_ref, hs_ref, dinv_ref, b_ref, g_ref, be_ref, w_ref,
              hbn_ref, hs2_ref):
    dinv = dinv_ref[...]
    a = aggp_ref[0, :N, :] + aggp_ref[1, :N, :] + hs_ref[...]
    pre = a * dinv + b_ref[...]
    mu = jnp.mean(pre, axis=0, keepdims=True)
    var = jnp.mean((pre - mu) ** 2, axis=0, keepdims=True)
    hbn = g_ref[...] * (pre - mu) / jnp.sqrt(var + 1e-5) + be_ref[...]
    hbn = jnp.maximum(hbn, 0.0)
    hbn_ref[...] = hbn
    hs2_ref[...] = jnp.dot(hbn, w_ref[...],
                           preferred_element_type=jnp.float32) * dinv


def _tcb(aggp, hs, dinv, b, g, be, w):
    return pl.pallas_call(
        _tcb_body,
        out_shape=(jax.ShapeDtypeStruct((N, D), jnp.float32),
                   jax.ShapeDtypeStruct((N, D), jnp.float32)),
    )(aggp, hs, dinv, b, g, be, w)


def _tcd_body(aggp_ref, hs_ref, dinv_ref, b_ref, out_ref):
    a = aggp_ref[0, :N, :] + aggp_ref[1, :N, :] + hs_ref[...]
    out_ref[...] = a * dinv_ref[...] + b_ref[...]


def _tcd(aggp, hs, dinv, b):
    return pl.pallas_call(
        _tcd_body,
        out_shape=jax.ShapeDtypeStruct((N, D), jnp.float32),
    )(aggp, hs, dinv, b)


def kernel(x, edge_index, W1, b1, g1, be1, W2, b2, g2, be2, W3, b3):
    src = edge_index[0].astype(jnp.int32)
    dst = edge_index[1].astype(jnp.int32)
    e = src.shape[0]
    pad = SLOTS - e
    srcI = jnp.concatenate([src, jnp.zeros((pad,), jnp.int32)])
    dstI = jnp.concatenate([dst, jnp.full((pad,), N, jnp.int32)])
    srcI = srcI.reshape(NW, NCH, CH)
    dstI = dstI.reshape(NW, NCH, CH)
    z1 = jnp.zeros((SROWS,), jnp.float32)
    z2 = jnp.zeros((CH, D), jnp.float32)

    degp = _deg_call(dstI, z1).reshape(NC, NPAD)
    dd = degp[:, :N].T
    h1s, dinv = _tca(dd, x, W1)

    aggp1 = _agg_call(h1s, srcI, dstI, z2)
    _, h2s = _tcb(aggp1, h1s, dinv, b1, g1, be1, W2)

    aggp2 = _agg_call(h2s, srcI, dstI, z2)
    h_out, h3s = _tcb(aggp2, h2s, dinv, b2, g2, be2, W3)

    aggp3 = _agg_call(h3s, srcI, dstI, z2)
    out = _tcd(aggp3, h3s, dinv, b3)
    return (h_out, out)
